# Initial kernel scaffold; baseline (speedup 1.0000x reference)
import jax
import jax.numpy as jnp
from jax.experimental import pallas as pl


def kernel(x, params, indices):
    raise NotImplementedError("write your pallas kernel here")


# speedup vs baseline: 10.6894x; 10.6894x over previous
import functools

import jax
import jax.numpy as jnp
from jax.experimental import pallas as pl
from jax.experimental.pallas import tpu as pltpu

_ROWS, _COLS = 16384, 4096
_BLOCK_ROWS = 512
_N_IDX = 2048


def _select_kernel(x_ref, p_ref, idx_ref, o_ref, mask_ref):
    @pl.when(pl.program_id(0) == 0)
    def _build_mask():
        iota = jax.lax.broadcasted_iota(jnp.int32, (8, _COLS), 1)

        def body(k, acc):
            chunk = idx_ref[pl.ds(k * 8, 8), :]
            return acc | (chunk == iota).astype(jnp.int32)

        acc = jax.lax.fori_loop(0, _N_IDX // 8, body,
                                jnp.zeros((8, _COLS), jnp.int32))
        mask_ref[...] = jnp.max(acc, axis=0, keepdims=True)

    t = p_ref[0, 0]
    xb = x_ref[...]
    m = mask_ref[...] != 0
    o_ref[...] = jnp.where(m, (xb > t).astype(xb.dtype), xb)


@functools.partial(jax.jit, static_argnames=())
def kernel(x, params, indices):
    idx2 = indices.reshape(_N_IDX, 1)
    p2 = params.reshape(1, 1)
    grid = _ROWS // _BLOCK_ROWS
    return pl.pallas_call(
        _select_kernel,
        grid=(grid,),
        in_specs=[
            pl.BlockSpec((_BLOCK_ROWS, _COLS), lambda i: (i, 0)),
            pl.BlockSpec((1, 1), lambda i: (0, 0)),
            pl.BlockSpec((_N_IDX, 1), lambda i: (0, 0)),
        ],
        out_specs=pl.BlockSpec((_BLOCK_ROWS, _COLS), lambda i: (i, 0)),
        out_shape=jax.ShapeDtypeStruct((_ROWS, _COLS), x.dtype),
        scratch_shapes=[pltpu.VMEM((1, _COLS), jnp.int32)],
    )(x, p2, idx2)


# speedup vs baseline: 10.7245x; 1.0033x over previous
import functools

import jax
import jax.numpy as jnp
from jax.experimental import pallas as pl
from jax.experimental.pallas import tpu as pltpu

_ROWS, _COLS = 16384, 4096
_BLOCK_ROWS = 512
_CHUNK_ROWS = 32
_N_IDX = 2048


def _select_kernel(x_ref, p_ref, idx_ref, o_ref, mask_ref):
    @pl.when(pl.program_id(0) == 0)
    def _build_mask():
        iota = jax.lax.broadcasted_iota(jnp.int32, (8, _COLS), 1)

        def body(k, acc):
            chunk = idx_ref[pl.ds(k * 8, 8), :]
            return acc | (chunk == iota).astype(jnp.int32)

        acc = jax.lax.fori_loop(0, _N_IDX // 8, body,
                                jnp.zeros((8, _COLS), jnp.int32))
        mask_ref[...] = jnp.max(acc, axis=0, keepdims=True)

    t = p_ref[0, 0]
    m = mask_ref[...] != 0

    def row_body(r, carry):
        xb = x_ref[pl.ds(r * _CHUNK_ROWS, _CHUNK_ROWS), :]
        o_ref[pl.ds(r * _CHUNK_ROWS, _CHUNK_ROWS), :] = jnp.where(
            m, (xb > t).astype(xb.dtype), xb)
        return carry

    jax.lax.fori_loop(0, _BLOCK_ROWS // _CHUNK_ROWS, row_body, 0)


@functools.partial(jax.jit, static_argnames=())
def kernel(x, params, indices):
    idx2 = indices.reshape(_N_IDX, 1)
    p2 = params.reshape(1, 1)
    grid = _ROWS // _BLOCK_ROWS
    return pl.pallas_call(
        _select_kernel,
        grid=(grid,),
        in_specs=[
            pl.BlockSpec((_BLOCK_ROWS, _COLS), lambda i: (i, 0)),
            pl.BlockSpec((1, 1), lambda i: (0, 0)),
            pl.BlockSpec((_N_IDX, 1), lambda i: (0, 0)),
        ],
        out_specs=pl.BlockSpec((_BLOCK_ROWS, _COLS), lambda i: (i, 0)),
        out_shape=jax.ShapeDtypeStruct((_ROWS, _COLS), x.dtype),
        scratch_shapes=[pltpu.VMEM((1, _COLS), jnp.int32)],
    )(x, p2, idx2)
